# baseline (speedup 1.0000x reference)
import functools

import jax
import jax.numpy as jnp
from jax import lax
from jax.experimental import pallas as pl
from jax.experimental.pallas import tpu as pltpu
from jax.experimental.pallas import tpu_sc as plsc

N = 10000
E = 320000
F_IN = 128
NHID = 128
NCLS = 64

NC, NS = 2, 16
NW = NC * NS
CHUNK = 64
CH = 160
EP = NW * CH * CHUNK
NP = 10240
RPT = NP // NS

_f32 = jnp.float32


def _mesh():
    return plsc.VectorSubcoreMesh(
        core_axis_name="c", subcore_axis_name="s",
        num_cores=NC, num_subcores=NS)


_EPT = CH * CHUNK


def _sc_degree(dst_flat):
    @functools.partial(
        pl.kernel,
        out_type=jax.ShapeDtypeStruct((NC, NP), _f32),
        mesh=_mesh(),
        compiler_params=pltpu.CompilerParams(needs_layout_passes=False),
        scratch_types=[
            pltpu.VMEM((_EPT,), jnp.int32),
            pltpu.VMEM((NP,), _f32),
            pltpu.VMEM((RPT,), _f32),
            pltpu.VMEM((RPT,), _f32),
            pltpu.VMEM_SHARED((NS, NP), _f32),
        ],
    )
    def k(dst_hbm, out, dst_v, hist, accv, tmpv, shared):
        c = lax.axis_index("c")
        s = lax.axis_index("s")
        wid = s * NC + c
        pltpu.sync_copy(dst_hbm.at[wid], dst_v)

        @pl.loop(0, NP // 16)
        def _(j):
            hist[pl.ds(j * 16, 16)] = jnp.zeros((16,), _f32)

        @pl.loop(0, _EPT // 16)
        def _(j):
            idx = dst_v[pl.ds(j * 16, 16)]
            cnt, last = plsc.scan_count(idx)
            plsc.addupdate_scatter(hist, [idx], cnt.astype(_f32), mask=last)

        pltpu.sync_copy(hist, shared.at[s])
        plsc.subcore_barrier()
        pltpu.sync_copy(shared.at[0, pl.ds(s * RPT, RPT)], accv)

        @pl.loop(1, NS)
        def _(t):
            pltpu.sync_copy(shared.at[t, pl.ds(s * RPT, RPT)], tmpv)

            @pl.loop(0, RPT // 16)
            def _(j):
                sl = pl.ds(j * 16, 16)
                accv[sl] = accv[sl] + tmpv[sl]

        pltpu.sync_copy(accv, out.at[c, pl.ds(s * RPT, RPT)])

    return k(dst_flat)


def _sc_scatter(h, src_r, dst_r, zeros_in, feat):
    @functools.partial(
        pl.kernel,
        out_type=jax.ShapeDtypeStruct((NC, NP, feat), _f32),
        mesh=_mesh(),
        scratch_types=[
            pltpu.VMEM((CH // 4, CHUNK), jnp.int32),
            pltpu.VMEM((CH // 4, CHUNK), jnp.int32),
            [pltpu.VMEM((CHUNK, feat), _f32) for _ in range(4)],
            [pltpu.SemaphoreType.DMA for _ in range(4)],
            [pltpu.SemaphoreType.DMA for _ in range(4)],
            pltpu.VMEM_SHARED((NP, feat), _f32),
        ],
    )
    def k(h_hbm, src_hbm, dst_hbm, z_hbm, out,
          src_v, dst_v, bufs, gsems, ssems, acc):
        c = lax.axis_index("c")
        s = lax.axis_index("s")
        wid = s * NC + c
        hch = CH // 4
        pltpu.sync_copy(z_hbm.at[pl.ds(s * RPT, RPT)],
                        acc.at[pl.ds(s * RPT, RPT)])
        plsc.subcore_barrier()

        def gather(i, j):
            pltpu.async_copy(h_hbm.at[src_v.at[i]], bufs[j], gsems[j])

        def gather_wait(i, j):
            pltpu.make_async_copy(h_hbm.at[src_v.at[i]], bufs[j],
                                  gsems[j]).wait()

        def scat(i, j):
            pltpu.async_copy(bufs[j], acc.at[dst_v.at[i]], ssems[j],
                             add=True)

        def scat_wait(i, j):
            pltpu.make_async_copy(bufs[j], acc.at[dst_v.at[i]],
                                  ssems[j]).wait()

        @pl.loop(0, 4)
        def _(half):
            pltpu.sync_copy(src_hbm.at[wid, pl.ds(half * hch, hch)], src_v)
            pltpu.sync_copy(dst_hbm.at[wid, pl.ds(half * hch, hch)], dst_v)
            gather(0, 0)
            gather(1, 1)

            @pl.loop(0, hch, step=4)
            def _(i):
                for jj in range(4):
                    ck = i + jj
                    j2 = (jj + 2) % 4
                    gather_wait(ck, jj)
                    scat(ck, jj)

                    @pl.when(ck + 2 < hch)
                    def _():
                        @pl.when(ck >= 2)
                        def _():
                            scat_wait(ck - 2, j2)

                        gather(ck + 2, j2)

            for jj in range(4):
                scat_wait(hch - 4 + jj, jj)

        plsc.subcore_barrier()
        pltpu.sync_copy(acc.at[pl.ds(s * RPT, RPT)],
                        out.at[c, pl.ds(s * RPT, RPT)])

    return k(h, src_r, dst_r, zeros_in)


_RB = 1024


def _dinv(a_ref, b_ref):
    return lax.rsqrt(a_ref[...] + b_ref[...] + 1.0)


_DSPEC0 = pl.BlockSpec((_RB, 1), lambda i: (i, 0))
_DSPEC1 = pl.BlockSpec((_RB, 1), lambda i: (i, 0))


def _tc_mm1(x, w1, d0c, d1c):
    def body(x_ref, w_ref, a_ref, b_ref, o_ref):
        o_ref[...] = _dinv(a_ref, b_ref) * jnp.dot(
            x_ref[...], w_ref[...], preferred_element_type=_f32)

    return pl.pallas_call(
        body,
        grid=(NP // _RB,),
        in_specs=[
            pl.BlockSpec((_RB, F_IN), lambda i: (i, 0)),
            pl.BlockSpec((F_IN, NHID), lambda i: (0, 0)),
            _DSPEC0,
            _DSPEC1,
        ],
        out_specs=pl.BlockSpec((_RB, NHID), lambda i: (i, 0)),
        out_shape=jax.ShapeDtypeStruct((NP, NHID), _f32),
    )(x, w1, d0c, d1c)


def _tc_mid(p, h1s, d0c, d1c, w2p, b1r):
    def body(p0r, p1r, hr, ar, br, wr, b1_, o_ref):
        dinv = _dinv(ar, br)
        t = (p0r[0] + p1r[0] + hr[...]) * dinv + b1_[...]
        h2 = jnp.maximum(t, 0.0)
        o_ref[...] = dinv * jnp.dot(h2, wr[...], preferred_element_type=_f32)

    return pl.pallas_call(
        body,
        grid=(NP // _RB,),
        in_specs=[
            pl.BlockSpec((1, _RB, NHID), lambda i: (0, i, 0)),
            pl.BlockSpec((1, _RB, NHID), lambda i: (1, i, 0)),
            pl.BlockSpec((_RB, NHID), lambda i: (i, 0)),
            _DSPEC0,
            _DSPEC1,
            pl.BlockSpec((NHID, NHID), lambda i: (0, 0)),
            pl.BlockSpec((1, NHID), lambda i: (0, 0)),
        ],
        out_specs=pl.BlockSpec((_RB, NHID), lambda i: (i, 0)),
        out_shape=jax.ShapeDtypeStruct((NP, NHID), _f32),
    )(p, p, h1s, d0c, d1c, w2p, b1r)


def _tc_fin(q, h2s, d0c, d1c, b2r):
    def body(q0r, q1r, hr, ar, br, b2_, o_ref):
        dinv = _dinv(ar, br)
        z = (q0r[0][:, :NCLS] + q1r[0][:, :NCLS] + hr[:, :NCLS]) * dinv \
            + b2_[...]
        m = jnp.max(z, axis=1, keepdims=True)
        ez = jnp.exp(z - m)
        o_ref[...] = z - (jnp.log(jnp.sum(ez, axis=1, keepdims=True)) + m)

    return pl.pallas_call(
        body,
        grid=(NP // _RB,),
        in_specs=[
            pl.BlockSpec((1, _RB, NHID), lambda i: (0, i, 0)),
            pl.BlockSpec((1, _RB, NHID), lambda i: (1, i, 0)),
            pl.BlockSpec((_RB, NHID), lambda i: (i, 0)),
            _DSPEC0,
            _DSPEC1,
            pl.BlockSpec((1, NCLS), lambda i: (0, 0)),
        ],
        out_specs=pl.BlockSpec((_RB, NCLS), lambda i: (i, 0)),
        out_shape=jax.ShapeDtypeStruct((NP, NCLS), _f32),
    )(q, q, h2s, d0c, d1c, b2r)


def kernel(x, edge_index, W1, b1, W2, b2):
    src = edge_index[0]
    dst = edge_index[1]
    pad = EP - E
    spread = jnp.arange(pad, dtype=jnp.int32) % (NP - N)
    srcp = jnp.concatenate([src, spread]).reshape(NW, CH, CHUNK)
    dstp = jnp.concatenate([dst, N + spread]).reshape(NW, CH, CHUNK)
    dst_flat = dstp.reshape(NW, _EPT)
    xp = jnp.pad(x, ((0, NP - N), (0, 0)))

    zeros_h = jnp.zeros((NP, NHID), _f32)
    w2p = jnp.pad(W2, ((0, 0), (0, NHID - NCLS)))

    dd = _sc_degree(dst_flat)
    d0c = dd[0].reshape(NP, 1)
    d1c = dd[1].reshape(NP, 1)
    h1s = _tc_mm1(xp, W1, d0c, d1c)
    p = _sc_scatter(h1s, srcp, dstp, zeros_h, NHID)
    h2s = _tc_mid(p, h1s, d0c, d1c, w2p, b1.reshape(1, NHID))
    q = _sc_scatter(h2s, srcp, dstp, zeros_h, NHID)
    outp = _tc_fin(q, h2s, d0c, d1c, b2.reshape(1, NCLS))
    return outp[:N]


# speedup vs baseline: 1.1036x; 1.1036x over previous
import functools

import jax
import jax.numpy as jnp
from jax import lax
from jax.experimental import pallas as pl
from jax.experimental.pallas import tpu as pltpu
from jax.experimental.pallas import tpu_sc as plsc

N = 10000
E = 320000
F_IN = 128
NHID = 128
NCLS = 64

NC, NS = 2, 16
NW = NC * NS
CHUNK = 128
CH = 80
EP = NW * CH * CHUNK
NP = 10240
RPT = NP // NS

_f32 = jnp.float32


def _mesh():
    return plsc.VectorSubcoreMesh(
        core_axis_name="c", subcore_axis_name="s",
        num_cores=NC, num_subcores=NS)


_EPT = CH * CHUNK


def _sc_degree(dst_flat):
    @functools.partial(
        pl.kernel,
        out_type=jax.ShapeDtypeStruct((NC, NP), _f32),
        mesh=_mesh(),
        compiler_params=pltpu.CompilerParams(needs_layout_passes=False),
        scratch_types=[
            pltpu.VMEM((_EPT,), jnp.int32),
            pltpu.VMEM((NP,), _f32),
            pltpu.VMEM((RPT,), _f32),
            pltpu.VMEM((RPT,), _f32),
            pltpu.VMEM_SHARED((NS, NP), _f32),
        ],
    )
    def k(dst_hbm, out, dst_v, hist, accv, tmpv, shared):
        c = lax.axis_index("c")
        s = lax.axis_index("s")
        wid = s * NC + c
        pltpu.sync_copy(dst_hbm.at[wid], dst_v)

        @pl.loop(0, NP // 16)
        def _(j):
            hist[pl.ds(j * 16, 16)] = jnp.zeros((16,), _f32)

        @pl.loop(0, _EPT // 16)
        def _(j):
            idx = dst_v[pl.ds(j * 16, 16)]
            cnt, last = plsc.scan_count(idx)
            plsc.addupdate_scatter(hist, [idx], cnt.astype(_f32), mask=last)

        pltpu.sync_copy(hist, shared.at[s])
        plsc.subcore_barrier()
        pltpu.sync_copy(shared.at[0, pl.ds(s * RPT, RPT)], accv)

        @pl.loop(1, NS)
        def _(t):
            pltpu.sync_copy(shared.at[t, pl.ds(s * RPT, RPT)], tmpv)

            @pl.loop(0, RPT // 16)
            def _(j):
                sl = pl.ds(j * 16, 16)
                accv[sl] = accv[sl] + tmpv[sl]

        pltpu.sync_copy(accv, out.at[c, pl.ds(s * RPT, RPT)])

    return k(dst_flat)


def _sc_scatter(h, src_r, dst_r, zeros_in, feat):
    @functools.partial(
        pl.kernel,
        out_type=jax.ShapeDtypeStruct((NC, NP, feat), _f32),
        mesh=_mesh(),
        scratch_types=[
            pltpu.VMEM((CH // 2, CHUNK), jnp.int32),
            pltpu.VMEM((CH // 2, CHUNK), jnp.int32),
            pltpu.VMEM((CHUNK, feat), _f32),
            pltpu.VMEM((CHUNK, feat), _f32),
            pltpu.VMEM_SHARED((NP, feat), _f32),
            pltpu.SemaphoreType.DMA,
            pltpu.SemaphoreType.DMA,
        ],
    )
    def k(h_hbm, src_hbm, dst_hbm, z_hbm, out,
          src_v, dst_v, buf_a, buf_b, acc, sem_a, sem_b):
        c = lax.axis_index("c")
        s = lax.axis_index("s")
        wid = s * NC + c
        hch = CH // 2
        pltpu.sync_copy(z_hbm.at[pl.ds(s * RPT, RPT)],
                        acc.at[pl.ds(s * RPT, RPT)])
        plsc.subcore_barrier()

        @pl.loop(0, 2)
        def _(half):
            pltpu.sync_copy(src_hbm.at[wid, pl.ds(half * hch, hch)], src_v)
            pltpu.sync_copy(dst_hbm.at[wid, pl.ds(half * hch, hch)], dst_v)
            pltpu.async_copy(h_hbm.at[src_v.at[0]], buf_a, sem_a)

            @pl.loop(0, hch, step=2)
            def _(i):
                pltpu.async_copy(h_hbm.at[src_v.at[i + 1]], buf_b, sem_b)
                pltpu.make_async_copy(h_hbm.at[src_v.at[i]], buf_a,
                                      sem_a).wait()
                pltpu.sync_copy(buf_a, acc.at[dst_v.at[i]], add=True)

                @pl.when(i + 2 < hch)
                def _():
                    pltpu.async_copy(h_hbm.at[src_v.at[i + 2]], buf_a, sem_a)

                pltpu.make_async_copy(h_hbm.at[src_v.at[i + 1]], buf_b,
                                      sem_b).wait()
                pltpu.sync_copy(buf_b, acc.at[dst_v.at[i + 1]], add=True)

        plsc.subcore_barrier()
        pltpu.sync_copy(acc.at[pl.ds(s * RPT, RPT)],
                        out.at[c, pl.ds(s * RPT, RPT)])

    return k(h, src_r, dst_r, zeros_in)


_RB = 1000


def _dinv(a_ref, b_ref):
    return lax.rsqrt(a_ref[...] + b_ref[...] + 1.0)


_DSPEC0 = pl.BlockSpec((_RB, 1), lambda i: (i, 0))
_DSPEC1 = pl.BlockSpec((_RB, 1), lambda i: (i, 0))


def _tc_mm1(x, w1, d0c, d1c):
    def body(x_ref, w_ref, a_ref, b_ref, o_ref):
        o_ref[...] = _dinv(a_ref, b_ref) * jnp.dot(
            x_ref[...], w_ref[...], preferred_element_type=_f32)

    return pl.pallas_call(
        body,
        grid=(N // _RB,),
        in_specs=[
            pl.BlockSpec((_RB, F_IN), lambda i: (i, 0)),
            pl.BlockSpec((F_IN, NHID), lambda i: (0, 0)),
            _DSPEC0,
            _DSPEC1,
        ],
        out_specs=pl.BlockSpec((_RB, NHID), lambda i: (i, 0)),
        out_shape=jax.ShapeDtypeStruct((N, NHID), _f32),
    )(x, w1, d0c, d1c)


def _tc_mid(p, h1s, d0c, d1c, w2p, b1r):
    def body(p0r, p1r, hr, ar, br, wr, b1_, o_ref):
        dinv = _dinv(ar, br)
        t = (p0r[0] + p1r[0] + hr[...]) * dinv + b1_[...]
        h2 = jnp.maximum(t, 0.0)
        o_ref[...] = dinv * jnp.dot(h2, wr[...], preferred_element_type=_f32)

    return pl.pallas_call(
        body,
        grid=(N // _RB,),
        in_specs=[
            pl.BlockSpec((1, _RB, NHID), lambda i: (0, i, 0)),
            pl.BlockSpec((1, _RB, NHID), lambda i: (1, i, 0)),
            pl.BlockSpec((_RB, NHID), lambda i: (i, 0)),
            _DSPEC0,
            _DSPEC1,
            pl.BlockSpec((NHID, NHID), lambda i: (0, 0)),
            pl.BlockSpec((1, NHID), lambda i: (0, 0)),
        ],
        out_specs=pl.BlockSpec((_RB, NHID), lambda i: (i, 0)),
        out_shape=jax.ShapeDtypeStruct((N, NHID), _f32),
    )(p, p, h1s, d0c, d1c, w2p, b1r)


def _tc_fin(q, h2s, d0c, d1c, b2r):
    def body(q0r, q1r, hr, ar, br, b2_, o_ref):
        dinv = _dinv(ar, br)
        z = (q0r[0][:, :NCLS] + q1r[0][:, :NCLS] + hr[:, :NCLS]) * dinv \
            + b2_[...]
        m = jnp.max(z, axis=1, keepdims=True)
        ez = jnp.exp(z - m)
        o_ref[...] = z - (jnp.log(jnp.sum(ez, axis=1, keepdims=True)) + m)

    return pl.pallas_call(
        body,
        grid=(N // _RB,),
        in_specs=[
            pl.BlockSpec((1, _RB, NHID), lambda i: (0, i, 0)),
            pl.BlockSpec((1, _RB, NHID), lambda i: (1, i, 0)),
            pl.BlockSpec((_RB, NHID), lambda i: (i, 0)),
            _DSPEC0,
            _DSPEC1,
            pl.BlockSpec((1, NCLS), lambda i: (0, 0)),
        ],
        out_specs=pl.BlockSpec((_RB, NCLS), lambda i: (i, 0)),
        out_shape=jax.ShapeDtypeStruct((N, NCLS), _f32),
    )(q, q, h2s, d0c, d1c, b2r)


def kernel(x, edge_index, W1, b1, W2, b2):
    src = edge_index[0]
    dst = edge_index[1]
    pad = EP - E
    spread = jnp.arange(pad, dtype=jnp.int32) % (NP - N)
    srcp = jnp.concatenate([src, spread]).reshape(NW, CH, CHUNK)
    dstp = jnp.concatenate([dst, N + spread]).reshape(NW, CH, CHUNK)
    dst_flat = dstp.reshape(NW, _EPT)

    zeros_h = jnp.zeros((NP, NHID), _f32)
    w2p = jnp.pad(W2, ((0, 0), (0, NHID - NCLS)))

    dd = _sc_degree(dst_flat)
    d0c = dd[0, :N].reshape(N, 1)
    d1c = dd[1, :N].reshape(N, 1)
    h1s = _tc_mm1(x, W1, d0c, d1c)
    p = _sc_scatter(h1s, srcp, dstp, zeros_h, NHID)
    h2s = _tc_mid(p, h1s, d0c, d1c, w2p, b1.reshape(1, NHID))
    q = _sc_scatter(h2s, srcp, dstp, zeros_h, NHID)
    return _tc_fin(q, h2s, d0c, d1c, b2.reshape(1, NCLS))


# speedup vs baseline: 1.1135x; 1.0090x over previous
import functools

import jax
import jax.numpy as jnp
from jax import lax
from jax.experimental import pallas as pl
from jax.experimental.pallas import tpu as pltpu
from jax.experimental.pallas import tpu_sc as plsc

N = 10000
E = 320000
F_IN = 128
NHID = 128
NCLS = 64

NC, NS = 2, 16
NW = NC * NS
CHUNK = 128
CH = 80
EP = NW * CH * CHUNK
NP = 10240
RPT = NP // NS

_f32 = jnp.float32


def _mesh():
    return plsc.VectorSubcoreMesh(
        core_axis_name="c", subcore_axis_name="s",
        num_cores=NC, num_subcores=NS)


_EPT = CH * CHUNK


def _sc_degree(dst_flat):
    @functools.partial(
        pl.kernel,
        out_type=jax.ShapeDtypeStruct((NC, NP), _f32),
        mesh=_mesh(),
        compiler_params=pltpu.CompilerParams(needs_layout_passes=False),
        scratch_types=[
            pltpu.VMEM((_EPT,), jnp.int32),
            pltpu.VMEM((NP,), _f32),
            pltpu.VMEM((RPT,), _f32),
            pltpu.VMEM((RPT,), _f32),
            pltpu.VMEM_SHARED((NS, NP), _f32),
        ],
    )
    def k(dst_hbm, out, dst_v, hist, accv, tmpv, shared):
        c = lax.axis_index("c")
        s = lax.axis_index("s")
        wid = s * NC + c
        pltpu.sync_copy(dst_hbm.at[wid], dst_v)

        @pl.loop(0, NP // 16, step=4)
        def _(j):
            for u in range(4):
                hist[pl.ds((j + u) * 16, 16)] = jnp.zeros((16,), _f32)

        @pl.loop(0, _EPT // 16, step=4)
        def _(j):
            for u in range(4):
                idx = dst_v[pl.ds((j + u) * 16, 16)]
                cnt, last = plsc.scan_count(idx)
                plsc.addupdate_scatter(hist, [idx], cnt.astype(_f32),
                                       mask=last)

        pltpu.sync_copy(hist, shared.at[s])
        plsc.subcore_barrier()
        pltpu.sync_copy(shared.at[0, pl.ds(s * RPT, RPT)], accv)

        @pl.loop(1, NS)
        def _(t):
            pltpu.sync_copy(shared.at[t, pl.ds(s * RPT, RPT)], tmpv)

            @pl.loop(0, RPT // 16)
            def _(j):
                sl = pl.ds(j * 16, 16)
                accv[sl] = accv[sl] + tmpv[sl]

        pltpu.sync_copy(accv, out.at[c, pl.ds(s * RPT, RPT)])

    return k(dst_flat)


def _sc_scatter(h, src_r, dst_r, zeros_in, feat):
    @functools.partial(
        pl.kernel,
        out_type=jax.ShapeDtypeStruct((NC, NP, feat), _f32),
        mesh=_mesh(),
        scratch_types=[
            pltpu.VMEM((CH // 2, CHUNK), jnp.int32),
            pltpu.VMEM((CH // 2, CHUNK), jnp.int32),
            pltpu.VMEM((CHUNK, feat), _f32),
            pltpu.VMEM((CHUNK, feat), _f32),
            pltpu.VMEM_SHARED((NP, feat), _f32),
            pltpu.SemaphoreType.DMA,
            pltpu.SemaphoreType.DMA,
        ],
    )
    def k(h_hbm, src_hbm, dst_hbm, z_hbm, out,
          src_v, dst_v, buf_a, buf_b, acc, sem_a, sem_b):
        c = lax.axis_index("c")
        s = lax.axis_index("s")
        wid = s * NC + c
        hch = CH // 2
        pltpu.sync_copy(z_hbm.at[pl.ds(s * RPT, RPT)],
                        acc.at[pl.ds(s * RPT, RPT)])
        plsc.subcore_barrier()

        @pl.loop(0, 2)
        def _(half):
            pltpu.sync_copy(src_hbm.at[wid, pl.ds(half * hch, hch)], src_v)
            pltpu.sync_copy(dst_hbm.at[wid, pl.ds(half * hch, hch)], dst_v)
            pltpu.async_copy(h_hbm.at[src_v.at[0]], buf_a, sem_a)

            @pl.loop(0, hch, step=2)
            def _(i):
                pltpu.async_copy(h_hbm.at[src_v.at[i + 1]], buf_b, sem_b)
                pltpu.make_async_copy(h_hbm.at[src_v.at[i]], buf_a,
                                      sem_a).wait()
                pltpu.sync_copy(buf_a, acc.at[dst_v.at[i]], add=True)

                @pl.when(i + 2 < hch)
                def _():
                    pltpu.async_copy(h_hbm.at[src_v.at[i + 2]], buf_a, sem_a)

                pltpu.make_async_copy(h_hbm.at[src_v.at[i + 1]], buf_b,
                                      sem_b).wait()
                pltpu.sync_copy(buf_b, acc.at[dst_v.at[i + 1]], add=True)

        plsc.subcore_barrier()
        pltpu.sync_copy(acc.at[pl.ds(s * RPT, RPT)],
                        out.at[c, pl.ds(s * RPT, RPT)])

    return k(h, src_r, dst_r, zeros_in)


_RB = 1000


def _dinv(a_ref, b_ref):
    return lax.rsqrt(a_ref[...] + b_ref[...] + 1.0)


_DSPEC0 = pl.BlockSpec((_RB, 1), lambda i: (i, 0))
_DSPEC1 = pl.BlockSpec((_RB, 1), lambda i: (i, 0))


def _tc_mm1(x, w1, d0c, d1c):
    def body(x_ref, w_ref, a_ref, b_ref, o_ref):
        o_ref[...] = _dinv(a_ref, b_ref) * jnp.dot(
            x_ref[...], w_ref[...], preferred_element_type=_f32)

    return pl.pallas_call(
        body,
        grid=(N // _RB,),
        in_specs=[
            pl.BlockSpec((_RB, F_IN), lambda i: (i, 0)),
            pl.BlockSpec((F_IN, NHID), lambda i: (0, 0)),
            _DSPEC0,
            _DSPEC1,
        ],
        out_specs=pl.BlockSpec((_RB, NHID), lambda i: (i, 0)),
        out_shape=jax.ShapeDtypeStruct((N, NHID), _f32),
    )(x, w1, d0c, d1c)


def _tc_mid(p, h1s, d0c, d1c, w2p, b1r):
    def body(p0r, p1r, hr, ar, br, wr, b1_, o_ref):
        dinv = _dinv(ar, br)
        t = (p0r[0] + p1r[0] + hr[...]) * dinv + b1_[...]
        h2 = jnp.maximum(t, 0.0)
        o_ref[...] = dinv * jnp.dot(h2, wr[...], preferred_element_type=_f32)

    return pl.pallas_call(
        body,
        grid=(N // _RB,),
        in_specs=[
            pl.BlockSpec((1, _RB, NHID), lambda i: (0, i, 0)),
            pl.BlockSpec((1, _RB, NHID), lambda i: (1, i, 0)),
            pl.BlockSpec((_RB, NHID), lambda i: (i, 0)),
            _DSPEC0,
            _DSPEC1,
            pl.BlockSpec((NHID, NHID), lambda i: (0, 0)),
            pl.BlockSpec((1, NHID), lambda i: (0, 0)),
        ],
        out_specs=pl.BlockSpec((_RB, NHID), lambda i: (i, 0)),
        out_shape=jax.ShapeDtypeStruct((N, NHID), _f32),
    )(p, p, h1s, d0c, d1c, w2p, b1r)


def _tc_fin(q, h2s, d0c, d1c, b2r):
    def body(q0r, q1r, hr, ar, br, b2_, o_ref):
        dinv = _dinv(ar, br)
        z = (q0r[0][:, :NCLS] + q1r[0][:, :NCLS] + hr[:, :NCLS]) * dinv \
            + b2_[...]
        m = jnp.max(z, axis=1, keepdims=True)
        ez = jnp.exp(z - m)
        o_ref[...] = z - (jnp.log(jnp.sum(ez, axis=1, keepdims=True)) + m)

    return pl.pallas_call(
        body,
        grid=(N // _RB,),
        in_specs=[
            pl.BlockSpec((1, _RB, NHID), lambda i: (0, i, 0)),
            pl.BlockSpec((1, _RB, NHID), lambda i: (1, i, 0)),
            pl.BlockSpec((_RB, NHID), lambda i: (i, 0)),
            _DSPEC0,
            _DSPEC1,
            pl.BlockSpec((1, NCLS), lambda i: (0, 0)),
        ],
        out_specs=pl.BlockSpec((_RB, NCLS), lambda i: (i, 0)),
        out_shape=jax.ShapeDtypeStruct((N, NCLS), _f32),
    )(q, q, h2s, d0c, d1c, b2r)


def kernel(x, edge_index, W1, b1, W2, b2):
    src = edge_index[0]
    dst = edge_index[1]
    pad = EP - E
    spread = jnp.arange(pad, dtype=jnp.int32) % (NP - N)
    srcp = jnp.concatenate([src, spread]).reshape(NW, CH, CHUNK)
    dstp = jnp.concatenate([dst, N + spread]).reshape(NW, CH, CHUNK)
    dst_flat = dstp.reshape(NW, _EPT)

    zeros_h = jnp.zeros((NP, NHID), _f32)
    w2p = jnp.pad(W2, ((0, 0), (0, NHID - NCLS)))

    dd = _sc_degree(dst_flat)
    d0c = dd[0, :N].reshape(N, 1)
    d1c = dd[1, :N].reshape(N, 1)
    h1s = _tc_mm1(x, W1, d0c, d1c)
    p = _sc_scatter(h1s, srcp, dstp, zeros_h, NHID)
    h2s = _tc_mid(p, h1s, d0c, d1c, w2p, b1.reshape(1, NHID))
    q = _sc_scatter(h2s, srcp, dstp, zeros_h, NHID)
    return _tc_fin(q, h2s, d0c, d1c, b2.reshape(1, NCLS))
